# baseline (speedup 1.0000x reference)
import functools
import jax
import jax.numpy as jnp
from jax import lax
from jax.experimental import pallas as pl
from jax.experimental.pallas import tpu as pltpu

_N_ITERS = 7
_NF = 10
_NI = 9
_EF = 11
_NO = 9
_H1 = 96
_PAD = 16

_EB = 12800
_NB = 2000

_INTERPRET = False


def _mlp_body(x_ref, w1_ref, b1_ref, w2_ref, b2_ref, w3_ref, b3_ref, out_ref):
    x = x_ref[...]
    m = jnp.maximum(
        jax.lax.dot_general(x, w1_ref[...], (((1,), (1,)), ((), ())),
                            preferred_element_type=jnp.float32) + b1_ref[...],
        0.0)
    m = jnp.maximum(
        jax.lax.dot_general(m, w2_ref[...], (((1,), (1,)), ((), ())),
                            preferred_element_type=jnp.float32) + b2_ref[...],
        0.0)
    out_ref[...] = jax.lax.dot_general(
        m, w3_ref[...], (((1,), (1,)), ((), ())),
        preferred_element_type=jnp.float32) + b3_ref[...]


def _edge_mlp(x, w1p, b1, w2, b2, w3p, b3p):
    n_edges = x.shape[0]
    grid = n_edges // _EB
    return pl.pallas_call(
        _mlp_body,
        grid=(grid,),
        in_specs=[
            pl.BlockSpec((_EB, 2 * _PAD), lambda i: (i, 0)),
            pl.BlockSpec((_H1, 2 * _PAD), lambda i: (0, 0)),
            pl.BlockSpec((1, _H1), lambda i: (0, 0)),
            pl.BlockSpec((_H1, _H1), lambda i: (0, 0)),
            pl.BlockSpec((1, _H1), lambda i: (0, 0)),
            pl.BlockSpec((_PAD, _H1), lambda i: (0, 0)),
            pl.BlockSpec((1, _PAD), lambda i: (0, 0)),
        ],
        out_specs=pl.BlockSpec((_EB, _PAD), lambda i: (i, 0)),
        out_shape=jax.ShapeDtypeStruct((n_edges, _PAD), jnp.float32),
        interpret=_INTERPRET,
    )(x, w1p, b1, w2, b2, w3p, b3p)


def _gru_body(ni_ref, agg_ref, h_ref, wia_ref, wib_ref, bih_ref, whh_ref,
              bhh_ref, wo_ref, bo_ref, hout_ref, out_ref):
    ni = ni_ref[...]
    agg = agg_ref[...]
    h = h_ref[...][:, :_NF]
    gi = (jax.lax.dot_general(ni, wia_ref[...], (((1,), (1,)), ((), ())),
                              preferred_element_type=jnp.float32)
          + jax.lax.dot_general(agg, wib_ref[...], (((1,), (1,)), ((), ())),
                                preferred_element_type=jnp.float32)
          + bih_ref[...])
    gh = jax.lax.dot_general(h, whh_ref[...], (((1,), (1,)), ((), ())),
                             preferred_element_type=jnp.float32) + bhh_ref[...]
    i_r = gi[:, :_NF]
    i_z = gi[:, _NF:2 * _NF]
    i_n = gi[:, 2 * _NF:]
    h_r = gh[:, :_NF]
    h_z = gh[:, _NF:2 * _NF]
    h_n = gh[:, 2 * _NF:]
    r = jax.nn.sigmoid(i_r + h_r)
    z = jax.nn.sigmoid(i_z + h_z)
    n = jnp.tanh(i_n + r * h_n)
    hn = (1.0 - z) * n + z * h
    hout_ref[...] = jnp.pad(hn, ((0, 0), (0, _PAD - _NF)))
    out_ref[...] = jax.lax.dot_general(
        hn, wo_ref[...], (((1,), (1,)), ((), ())),
        preferred_element_type=jnp.float32) + bo_ref[...]


def _gru_step(node_inputs, agg, h_pad, wia, wib, b_ih, w_hh, b_hh, wo, bo):
    n_nodes = node_inputs.shape[0]
    grid = n_nodes // _NB
    return pl.pallas_call(
        _gru_body,
        grid=(grid,),
        in_specs=[
            pl.BlockSpec((_NB, _NI), lambda i: (i, 0)),
            pl.BlockSpec((_NB, _PAD), lambda i: (i, 0)),
            pl.BlockSpec((_NB, _PAD), lambda i: (i, 0)),
            pl.BlockSpec((3 * _NF, _NI), lambda i: (0, 0)),
            pl.BlockSpec((3 * _NF, _PAD), lambda i: (0, 0)),
            pl.BlockSpec((1, 3 * _NF), lambda i: (0, 0)),
            pl.BlockSpec((3 * _NF, _NF), lambda i: (0, 0)),
            pl.BlockSpec((1, 3 * _NF), lambda i: (0, 0)),
            pl.BlockSpec((_NO, _NF), lambda i: (0, 0)),
            pl.BlockSpec((1, _NO), lambda i: (0, 0)),
        ],
        out_specs=[
            pl.BlockSpec((_NB, _PAD), lambda i: (i, 0)),
            pl.BlockSpec((_NB, _NO), lambda i: (i, 0)),
        ],
        out_shape=[
            jax.ShapeDtypeStruct((n_nodes, _PAD), jnp.float32),
            jax.ShapeDtypeStruct((n_nodes, _NO), jnp.float32),
        ],
        interpret=_INTERPRET,
    )(node_inputs, agg, h_pad, wia, wib, b_ih, w_hh, b_hh, wo, bo)


def kernel(node_inputs, src_ids, dst_ids, W1, b1, W2, b2, W3, b3,
           W_ih, b_ih, W_hh, b_hh, Wo, bo):
    n_nodes = node_inputs.shape[0]
    n_edges = src_ids.shape[0]

    idx2 = jnp.stack([src_ids, dst_ids], axis=1).reshape(-1)

    w1p = jnp.zeros((_H1, 2 * _PAD), jnp.float32)
    w1p = w1p.at[:, :_NF].set(W1[:, :_NF]).at[:, _PAD:_PAD + _NF].set(W1[:, _NF:])
    w3p = jnp.zeros((_PAD, _H1), jnp.float32).at[:_EF, :].set(W3)
    b3p = jnp.zeros((1, _PAD), jnp.float32).at[0, :_EF].set(b3)
    wia = W_ih[:, :_NI]
    wib = jnp.zeros((3 * _NF, _PAD), jnp.float32).at[:, :_EF].set(W_ih[:, _NI:])

    b1r = b1.reshape(1, -1)
    b2r = b2.reshape(1, -1)
    bihr = b_ih.reshape(1, -1)
    bhhr = b_hh.reshape(1, -1)
    bor = bo.reshape(1, -1)

    h_pad = jnp.zeros((n_nodes, _PAD), jnp.float32)
    outputs = []
    for _ in range(_N_ITERS):
        gathered = h_pad[idx2].reshape(n_edges, 2 * _PAD)
        messages = _edge_mlp(gathered, w1p, b1r, W2, b2r, w3p, b3p)
        agg = jax.ops.segment_sum(messages, dst_ids, num_segments=n_nodes)
        h_pad, out_t = _gru_step(node_inputs, agg, h_pad, wia, wib, bihr,
                                 W_hh, bhhr, Wo, bor)
        outputs.append(out_t)
    return jnp.stack(outputs, axis=0)


# speedup vs baseline: 3.1096x; 3.1096x over previous
import functools
import jax
import jax.numpy as jnp
from jax import lax
from jax.experimental import pallas as pl
from jax.experimental.pallas import tpu as pltpu
from jax.experimental.pallas import tpu_sc as plsc

_N_ITERS = 7
_NF = 10
_NI = 9
_EF = 11
_NO = 9
_H1 = 96
_PAD = 16

_EB = 12800
_NB = 2000

_NC = 2
_NS = 16
_NW = _NC * _NS
_GT = 125
_GK = 8
_GCH = _GT * _GK

_INTERPRET = False


def _sc_gather(h_pad, src2d, dst2d, n_edges):
    per_w = n_edges // _NW
    n_chunks = per_w // _GCH
    rows_per_chunk = _GK
    mesh = plsc.VectorSubcoreMesh(core_axis_name="c", subcore_axis_name="s")

    @functools.partial(
        pl.kernel,
        mesh=mesh,
        compiler_params=pltpu.CompilerParams(use_tc_tiling_on_sc=False),
        out_type=[
            jax.ShapeDtypeStruct((n_edges, _PAD), jnp.float32),
            jax.ShapeDtypeStruct((n_edges, _PAD), jnp.float32),
        ],
        scratch_types=[
            pltpu.VMEM((_GK, _GT), jnp.int32),
            pltpu.VMEM((_GCH, _PAD), jnp.float32),
            pltpu.SemaphoreType.DMA,
        ],
    )
    def gather_kernel(h_hbm, src_hbm, dst_hbm, osrc_hbm, odst_hbm,
                      idxv, rows, sem):
        wid = lax.axis_index("s") * _NC + lax.axis_index("c")

        def run(idx_hbm, out_hbm):
            def chunk(ci, carry):
                row0 = wid * (per_w // _GT) + ci * rows_per_chunk
                pltpu.sync_copy(idx_hbm.at[pl.ds(row0, rows_per_chunk)], idxv)
                cps = [
                    pltpu.async_copy(h_hbm.at[idxv.at[j]],
                                     rows.at[pl.ds(j * _GT, _GT)], sem)
                    for j in range(_GK)
                ]
                for c in cps:
                    c.wait()
                base = wid * per_w + ci * _GCH
                pltpu.sync_copy(rows, out_hbm.at[pl.ds(base, _GCH)])
                return carry
            lax.fori_loop(0, n_chunks, chunk, 0)

        run(src_hbm, osrc_hbm)
        run(dst_hbm, odst_hbm)

    return gather_kernel(h_pad, src2d, dst2d)


def _mlp_body(xs_ref, xd_ref, w1s_ref, w1d_ref, b1_ref, w2_ref, b2_ref,
              w3_ref, b3_ref, out_ref):
    dn = (((1,), (1,)), ((), ()))
    m = jnp.maximum(
        jax.lax.dot_general(xs_ref[...], w1s_ref[...], dn,
                            preferred_element_type=jnp.float32)
        + jax.lax.dot_general(xd_ref[...], w1d_ref[...], dn,
                              preferred_element_type=jnp.float32)
        + b1_ref[...], 0.0)
    m = jnp.maximum(
        jax.lax.dot_general(m, w2_ref[...], dn,
                            preferred_element_type=jnp.float32) + b2_ref[...],
        0.0)
    out_ref[...] = jax.lax.dot_general(
        m, w3_ref[...], dn, preferred_element_type=jnp.float32) + b3_ref[...]


def _edge_mlp(xs, xd, w1s, w1d, b1, w2, b2, w3p, b3p):
    n_edges = xs.shape[0]
    grid = n_edges // _EB
    return pl.pallas_call(
        _mlp_body,
        grid=(grid,),
        in_specs=[
            pl.BlockSpec((_EB, _PAD), lambda i: (i, 0)),
            pl.BlockSpec((_EB, _PAD), lambda i: (i, 0)),
            pl.BlockSpec((_H1, _PAD), lambda i: (0, 0)),
            pl.BlockSpec((_H1, _PAD), lambda i: (0, 0)),
            pl.BlockSpec((1, _H1), lambda i: (0, 0)),
            pl.BlockSpec((_H1, _H1), lambda i: (0, 0)),
            pl.BlockSpec((1, _H1), lambda i: (0, 0)),
            pl.BlockSpec((_PAD, _H1), lambda i: (0, 0)),
            pl.BlockSpec((1, _PAD), lambda i: (0, 0)),
        ],
        out_specs=pl.BlockSpec((_EB, _PAD), lambda i: (i, 0)),
        out_shape=jax.ShapeDtypeStruct((n_edges, _PAD), jnp.float32),
        interpret=_INTERPRET,
    )(xs, xd, w1s, w1d, b1, w2, b2, w3p, b3p)


def _gru_body(ni_ref, agg_ref, h_ref, wia_ref, wib_ref, bih_ref, whh_ref,
              bhh_ref, wo_ref, bo_ref, hout_ref, out_ref):
    dn = (((1,), (1,)), ((), ()))
    ni = ni_ref[...]
    agg = agg_ref[...]
    h = h_ref[...][:, :_NF]
    gi = (jax.lax.dot_general(ni, wia_ref[...], dn,
                              preferred_element_type=jnp.float32)
          + jax.lax.dot_general(agg, wib_ref[...], dn,
                                preferred_element_type=jnp.float32)
          + bih_ref[...])
    gh = jax.lax.dot_general(h, whh_ref[...], dn,
                             preferred_element_type=jnp.float32) + bhh_ref[...]
    i_r = gi[:, :_NF]
    i_z = gi[:, _NF:2 * _NF]
    i_n = gi[:, 2 * _NF:]
    h_r = gh[:, :_NF]
    h_z = gh[:, _NF:2 * _NF]
    h_n = gh[:, 2 * _NF:]
    r = jax.nn.sigmoid(i_r + h_r)
    z = jax.nn.sigmoid(i_z + h_z)
    n = jnp.tanh(i_n + r * h_n)
    hn = (1.0 - z) * n + z * h
    hout_ref[...] = jnp.pad(hn, ((0, 0), (0, _PAD - _NF)))
    out_ref[...] = jax.lax.dot_general(
        hn, wo_ref[...], dn, preferred_element_type=jnp.float32) + bo_ref[...]


def _gru_step(node_inputs, agg, h_pad, wia, wib, b_ih, w_hh, b_hh, wo, bo):
    n_nodes = node_inputs.shape[0]
    grid = n_nodes // _NB
    return pl.pallas_call(
        _gru_body,
        grid=(grid,),
        in_specs=[
            pl.BlockSpec((_NB, _NI), lambda i: (i, 0)),
            pl.BlockSpec((_NB, _PAD), lambda i: (i, 0)),
            pl.BlockSpec((_NB, _PAD), lambda i: (i, 0)),
            pl.BlockSpec((3 * _NF, _NI), lambda i: (0, 0)),
            pl.BlockSpec((3 * _NF, _PAD), lambda i: (0, 0)),
            pl.BlockSpec((1, 3 * _NF), lambda i: (0, 0)),
            pl.BlockSpec((3 * _NF, _NF), lambda i: (0, 0)),
            pl.BlockSpec((1, 3 * _NF), lambda i: (0, 0)),
            pl.BlockSpec((_NO, _NF), lambda i: (0, 0)),
            pl.BlockSpec((1, _NO), lambda i: (0, 0)),
        ],
        out_specs=[
            pl.BlockSpec((_NB, _PAD), lambda i: (i, 0)),
            pl.BlockSpec((_NB, _NO), lambda i: (i, 0)),
        ],
        out_shape=[
            jax.ShapeDtypeStruct((n_nodes, _PAD), jnp.float32),
            jax.ShapeDtypeStruct((n_nodes, _NO), jnp.float32),
        ],
        interpret=_INTERPRET,
    )(node_inputs, agg, h_pad, wia, wib, b_ih, w_hh, b_hh, wo, bo)


def kernel(node_inputs, src_ids, dst_ids, W1, b1, W2, b2, W3, b3,
           W_ih, b_ih, W_hh, b_hh, Wo, bo):
    n_nodes = node_inputs.shape[0]
    n_edges = src_ids.shape[0]

    src2d = src_ids.reshape(n_edges // _GT, _GT)
    dst2d = dst_ids.reshape(n_edges // _GT, _GT)

    w1s = jnp.zeros((_H1, _PAD), jnp.float32).at[:, :_NF].set(W1[:, :_NF])
    w1d = jnp.zeros((_H1, _PAD), jnp.float32).at[:, :_NF].set(W1[:, _NF:])
    w3p = jnp.zeros((_PAD, _H1), jnp.float32).at[:_EF, :].set(W3)
    b3p = jnp.zeros((1, _PAD), jnp.float32).at[0, :_EF].set(b3)
    wia = W_ih[:, :_NI]
    wib = jnp.zeros((3 * _NF, _PAD), jnp.float32).at[:, :_EF].set(W_ih[:, _NI:])

    b1r = b1.reshape(1, -1)
    b2r = b2.reshape(1, -1)
    bihr = b_ih.reshape(1, -1)
    bhhr = b_hh.reshape(1, -1)
    bor = bo.reshape(1, -1)

    h_pad = jnp.zeros((n_nodes, _PAD), jnp.float32)
    outputs = []
    for _ in range(_N_ITERS):
        xs, xd = _sc_gather(h_pad, src2d, dst2d, n_edges)
        messages = _edge_mlp(xs, xd, w1s, w1d, b1r, W2, b2r, w3p, b3p)
        agg = jax.ops.segment_sum(messages, dst_ids, num_segments=n_nodes)
        h_pad, out_t = _gru_step(node_inputs, agg, h_pad, wia, wib, bihr,
                                 W_hh, bhhr, Wo, bor)
        outputs.append(out_t)
    return jnp.stack(outputs, axis=0)


# speedup vs baseline: 5.2352x; 1.6836x over previous
import functools
import jax
import jax.numpy as jnp
from jax import lax
from jax.experimental import pallas as pl
from jax.experimental.pallas import tpu as pltpu
from jax.experimental.pallas import tpu_sc as plsc

_N_ITERS = 7
_NF = 10
_NI = 9
_EF = 11
_NO = 9
_H1 = 96
_PAD = 16

_EB = 12800
_NB = 2000

_NC = 2
_NS = 16
_NW = _NC * _NS
_GT = 125
_GK = 8
_GCH = _GT * _GK

_INTERPRET = False


def _sc_gather(h_pad, src2d, dst2d, n_edges):
    per_w = n_edges // _NW
    n_chunks = per_w // _GCH
    rows_per_chunk = _GK
    mesh = plsc.VectorSubcoreMesh(core_axis_name="c", subcore_axis_name="s")

    @functools.partial(
        pl.kernel,
        mesh=mesh,
        compiler_params=pltpu.CompilerParams(use_tc_tiling_on_sc=False),
        out_type=[
            jax.ShapeDtypeStruct((n_edges, _PAD), jnp.float32),
            jax.ShapeDtypeStruct((n_edges, _PAD), jnp.float32),
        ],
        scratch_types=[
            pltpu.VMEM((_GK, _GT), jnp.int32),
            pltpu.VMEM((_GCH, _PAD), jnp.float32),
            pltpu.SemaphoreType.DMA,
        ],
    )
    def gather_kernel(h_hbm, src_hbm, dst_hbm, osrc_hbm, odst_hbm,
                      idxv, rows, sem):
        wid = lax.axis_index("s") * _NC + lax.axis_index("c")

        def run(idx_hbm, out_hbm):
            def chunk(ci, carry):
                row0 = wid * (per_w // _GT) + ci * rows_per_chunk
                pltpu.sync_copy(idx_hbm.at[pl.ds(row0, rows_per_chunk)], idxv)
                cps = [
                    pltpu.async_copy(h_hbm.at[idxv.at[j]],
                                     rows.at[pl.ds(j * _GT, _GT)], sem)
                    for j in range(_GK)
                ]
                for c in cps:
                    c.wait()
                base = wid * per_w + ci * _GCH
                pltpu.sync_copy(rows, out_hbm.at[pl.ds(base, _GCH)])
                return carry
            lax.fori_loop(0, n_chunks, chunk, 0)

        run(src_hbm, osrc_hbm)
        run(dst_hbm, odst_hbm)

    return gather_kernel(h_pad, src2d, dst2d)


_ST = 64
_SR = 8
_SCH = _ST * _SR
_HALF = 50000
_TRASH = 2048
_AGGR = _HALF + _TRASH


def _sc_scatter(messages, dst64, n_nodes, n_edges):
    n_chunks = n_edges // _SCH
    k_max = (n_chunks + _NS - 1) // _NS
    zb = 2000
    n_zchunks = _AGGR // zb
    n_ochunks = _HALF // zb
    mesh = plsc.VectorSubcoreMesh(core_axis_name="c", subcore_axis_name="s")

    @functools.partial(
        pl.kernel,
        mesh=mesh,
        compiler_params=pltpu.CompilerParams(use_tc_tiling_on_sc=False),
        out_type=jax.ShapeDtypeStruct((_NC, _HALF, _PAD), jnp.float32),
        scratch_types=[
            pltpu.VMEM((_SR, _ST), jnp.int32),
            pltpu.VMEM((_SR, _ST), jnp.int32),
            pltpu.VMEM((_SCH, _PAD), jnp.float32),
            pltpu.VMEM((zb, _PAD), jnp.float32),
            pltpu.VMEM_SHARED((_AGGR, _PAD), jnp.float32),
            pltpu.SemaphoreType.DMA,
        ],
    )
    def scatter_kernel(msg_hbm, dst_hbm, out_hbm, idxv, idxw, msgb, obuf,
                       aggsh, sem):
        cid = lax.axis_index("c")
        sid = lax.axis_index("s")
        lo = cid * _HALF

        def zrow(i, carry):
            obuf[i, :] = jnp.zeros((_PAD,), jnp.float32)
            return carry
        lax.fori_loop(0, zb, zrow, 0)

        def zchunk(k, carry):
            c = sid + _NS * k

            @pl.when(c < n_zchunks)
            def _():
                pltpu.sync_copy(obuf, aggsh.at[pl.ds(c * zb, zb)])
            return carry
        lax.fori_loop(0, (n_zchunks + _NS - 1) // _NS, zchunk, 0)
        plsc.subcore_barrier()

        def chunk(k, carry):
            c = sid + _NS * k

            @pl.when(c < n_chunks)
            def _():
                pltpu.sync_copy(dst_hbm.at[pl.ds(c * _SR, _SR)], idxv)
                pltpu.sync_copy(msg_hbm.at[pl.ds(c * _SCH, _SCH)], msgb)
                for j in range(_SR):
                    for t in range(_ST // 16):
                        v = idxv[j, pl.ds(t * 16, 16)]
                        local = v - lo
                        ok = (local >= 0) & (local < _HALF)
                        idxw[j, pl.ds(t * 16, 16)] = jnp.where(
                            ok, local, _HALF + (v & (_TRASH - 1)))
                for j in range(_SR):
                    pltpu.sync_copy(msgb.at[pl.ds(j * _ST, _ST)],
                                    aggsh.at[idxw.at[j]], add=True)
            return carry
        lax.fori_loop(0, k_max, chunk, 0)
        plsc.subcore_barrier()

        def ochunk(k, carry):
            c = sid + _NS * k

            @pl.when(c < n_ochunks)
            def _():
                pltpu.sync_copy(aggsh.at[pl.ds(c * zb, zb)], obuf)
                pltpu.sync_copy(obuf, out_hbm.at[cid].at[pl.ds(c * zb, zb)])
            return carry
        lax.fori_loop(0, (n_ochunks + _NS - 1) // _NS, ochunk, 0)

    return scatter_kernel(messages, dst64)


def _mlp_body(xs_ref, xd_ref, w1s_ref, w1d_ref, b1_ref, w2_ref, b2_ref,
              w3_ref, b3_ref, out_ref):
    dn = (((1,), (1,)), ((), ()))
    m = jnp.maximum(
        jax.lax.dot_general(xs_ref[...], w1s_ref[...], dn,
                            preferred_element_type=jnp.float32)
        + jax.lax.dot_general(xd_ref[...], w1d_ref[...], dn,
                              preferred_element_type=jnp.float32)
        + b1_ref[...], 0.0)
    m = jnp.maximum(
        jax.lax.dot_general(m, w2_ref[...], dn,
                            preferred_element_type=jnp.float32) + b2_ref[...],
        0.0)
    out_ref[...] = jax.lax.dot_general(
        m, w3_ref[...], dn, preferred_element_type=jnp.float32) + b3_ref[...]


def _edge_mlp(xs, xd, w1s, w1d, b1, w2, b2, w3p, b3p):
    n_edges = xs.shape[0]
    grid = n_edges // _EB
    return pl.pallas_call(
        _mlp_body,
        grid=(grid,),
        in_specs=[
            pl.BlockSpec((_EB, _PAD), lambda i: (i, 0)),
            pl.BlockSpec((_EB, _PAD), lambda i: (i, 0)),
            pl.BlockSpec((_H1, _PAD), lambda i: (0, 0)),
            pl.BlockSpec((_H1, _PAD), lambda i: (0, 0)),
            pl.BlockSpec((1, _H1), lambda i: (0, 0)),
            pl.BlockSpec((_H1, _H1), lambda i: (0, 0)),
            pl.BlockSpec((1, _H1), lambda i: (0, 0)),
            pl.BlockSpec((_PAD, _H1), lambda i: (0, 0)),
            pl.BlockSpec((1, _PAD), lambda i: (0, 0)),
        ],
        out_specs=pl.BlockSpec((_EB, _PAD), lambda i: (i, 0)),
        out_shape=jax.ShapeDtypeStruct((n_edges, _PAD), jnp.float32),
        interpret=_INTERPRET,
    )(xs, xd, w1s, w1d, b1, w2, b2, w3p, b3p)


def _gru_body(ni_ref, agg_ref, h_ref, wia_ref, wib_ref, bih_ref,
              whh_ref, bhh_ref, wo_ref, bo_ref, hout_ref, out_ref):
    dn = (((1,), (1,)), ((), ()))
    ni = ni_ref[...]
    agg = agg_ref[...]
    h = h_ref[...][:, :_NF]
    gi = (jax.lax.dot_general(ni, wia_ref[...], dn,
                              preferred_element_type=jnp.float32)
          + jax.lax.dot_general(agg, wib_ref[...], dn,
                                preferred_element_type=jnp.float32)
          + bih_ref[...])
    gh = jax.lax.dot_general(h, whh_ref[...], dn,
                             preferred_element_type=jnp.float32) + bhh_ref[...]
    i_r = gi[:, :_NF]
    i_z = gi[:, _NF:2 * _NF]
    i_n = gi[:, 2 * _NF:]
    h_r = gh[:, :_NF]
    h_z = gh[:, _NF:2 * _NF]
    h_n = gh[:, 2 * _NF:]
    r = jax.nn.sigmoid(i_r + h_r)
    z = jax.nn.sigmoid(i_z + h_z)
    n = jnp.tanh(i_n + r * h_n)
    hn = (1.0 - z) * n + z * h
    hout_ref[...] = jnp.pad(hn, ((0, 0), (0, _PAD - _NF)))
    out_ref[...] = jax.lax.dot_general(
        hn, wo_ref[...], dn, preferred_element_type=jnp.float32) + bo_ref[...]


def _gru_step(node_inputs, aggp, h_pad, wia, wib, b_ih, w_hh, b_hh, wo, bo):
    n_nodes = node_inputs.shape[0]
    grid = n_nodes // _NB
    return pl.pallas_call(
        _gru_body,
        grid=(grid,),
        in_specs=[
            pl.BlockSpec((_NB, _NI), lambda i: (i, 0)),
            pl.BlockSpec((_NB, _PAD), lambda i: (i, 0)),
            pl.BlockSpec((_NB, _PAD), lambda i: (i, 0)),
            pl.BlockSpec((3 * _NF, _NI), lambda i: (0, 0)),
            pl.BlockSpec((3 * _NF, _PAD), lambda i: (0, 0)),
            pl.BlockSpec((1, 3 * _NF), lambda i: (0, 0)),
            pl.BlockSpec((3 * _NF, _NF), lambda i: (0, 0)),
            pl.BlockSpec((1, 3 * _NF), lambda i: (0, 0)),
            pl.BlockSpec((_NO, _NF), lambda i: (0, 0)),
            pl.BlockSpec((1, _NO), lambda i: (0, 0)),
        ],
        out_specs=[
            pl.BlockSpec((_NB, _PAD), lambda i: (i, 0)),
            pl.BlockSpec((_NB, _NO), lambda i: (i, 0)),
        ],
        out_shape=[
            jax.ShapeDtypeStruct((n_nodes, _PAD), jnp.float32),
            jax.ShapeDtypeStruct((n_nodes, _NO), jnp.float32),
        ],
        interpret=_INTERPRET,
    )(node_inputs, aggp, h_pad, wia, wib, b_ih, w_hh, b_hh, wo, bo)


def kernel(node_inputs, src_ids, dst_ids, W1, b1, W2, b2, W3, b3,
           W_ih, b_ih, W_hh, b_hh, Wo, bo):
    n_nodes = node_inputs.shape[0]
    n_edges = src_ids.shape[0]

    src2d = src_ids.reshape(n_edges // _GT, _GT)
    dst2d = dst_ids.reshape(n_edges // _GT, _GT)
    dst64 = dst_ids.reshape(n_edges // _ST, _ST)

    w1s = jnp.zeros((_H1, _PAD), jnp.float32).at[:, :_NF].set(W1[:, :_NF])
    w1d = jnp.zeros((_H1, _PAD), jnp.float32).at[:, :_NF].set(W1[:, _NF:])
    w3p = jnp.zeros((_PAD, _H1), jnp.float32).at[:_EF, :].set(W3)
    b3p = jnp.zeros((1, _PAD), jnp.float32).at[0, :_EF].set(b3)
    wia = W_ih[:, :_NI]
    wib = jnp.zeros((3 * _NF, _PAD), jnp.float32).at[:, :_EF].set(W_ih[:, _NI:])

    b1r = b1.reshape(1, -1)
    b2r = b2.reshape(1, -1)
    bihr = b_ih.reshape(1, -1)
    bhhr = b_hh.reshape(1, -1)
    bor = bo.reshape(1, -1)

    h_pad = jnp.zeros((n_nodes, _PAD), jnp.float32)
    outputs = []
    for _ in range(_N_ITERS):
        xs, xd = _sc_gather(h_pad, src2d, dst2d, n_edges)
        messages = _edge_mlp(xs, xd, w1s, w1d, b1r, W2, b2r, w3p, b3p)
        aggp = _sc_scatter(messages, dst64, n_nodes, n_edges)
        agg = aggp.reshape(n_nodes, _PAD)
        h_pad, out_t = _gru_step(node_inputs, agg, h_pad, wia, wib, bihr,
                                 W_hh, bhhr, Wo, bor)
        outputs.append(out_t)
    return jnp.stack(outputs, axis=0)


# speedup vs baseline: 5.8542x; 1.1182x over previous
import functools
import jax
import jax.numpy as jnp
from jax import lax
from jax.experimental import pallas as pl
from jax.experimental.pallas import tpu as pltpu
from jax.experimental.pallas import tpu_sc as plsc

_N_ITERS = 7
_NF = 10
_NI = 9
_EF = 11
_NO = 9
_H1 = 96
_PAD = 16

_EB = 12800
_NB = 2000

_NC = 2
_NS = 16
_NW = _NC * _NS
_GT = 125
_GK = 8
_GCH = _GT * _GK

_INTERPRET = False


def _sc_gather(h_pad, src2d, dst2d, n_edges):
    per_w = n_edges // _NW
    n_chunks = per_w // _GCH
    rows_per_chunk = _GK
    mesh = plsc.VectorSubcoreMesh(core_axis_name="c", subcore_axis_name="s")

    @functools.partial(
        pl.kernel,
        mesh=mesh,
        compiler_params=pltpu.CompilerParams(use_tc_tiling_on_sc=False),
        out_type=[
            jax.ShapeDtypeStruct((n_edges, _PAD), jnp.float32),
            jax.ShapeDtypeStruct((n_edges, _PAD), jnp.float32),
        ],
        scratch_types=[
            pltpu.VMEM((2, _GK, _GT), jnp.int32),
            pltpu.VMEM((2, _GCH, _PAD), jnp.float32),
            pltpu.SemaphoreType.DMA,
            pltpu.SemaphoreType.DMA,
            pltpu.SemaphoreType.DMA,
        ],
    )
    def gather_kernel(h_hbm, src_hbm, dst_hbm, osrc_hbm, odst_hbm,
                      idxv, rows, semi, semg, semo):
        wid = lax.axis_index("s") * _NC + lax.axis_index("c")
        row_base = wid * (per_w // _GT)
        out_base = wid * per_w

        def run(idx_hbm, out_hbm):
            pltpu.async_copy(idx_hbm.at[pl.ds(row_base, rows_per_chunk)],
                             idxv.at[0], semi)

            def chunk(ci, carry):
                b = lax.rem(ci, 2)
                pltpu.make_async_copy(
                    idx_hbm.at[pl.ds(row_base, rows_per_chunk)],
                    idxv.at[b], semi).wait()

                @pl.when(ci + 1 < n_chunks)
                def _():
                    pltpu.async_copy(
                        idx_hbm.at[pl.ds(row_base + (ci + 1) * rows_per_chunk,
                                         rows_per_chunk)],
                        idxv.at[1 - b], semi)

                @pl.when(ci >= 2)
                def _():
                    pltpu.make_async_copy(
                        rows.at[b],
                        out_hbm.at[pl.ds(out_base, _GCH)], semo).wait()

                cps = [
                    pltpu.async_copy(h_hbm.at[idxv.at[b, j]],
                                     rows.at[b, pl.ds(j * _GT, _GT)], semg)
                    for j in range(_GK)
                ]
                for c in cps:
                    c.wait()
                pltpu.async_copy(rows.at[b],
                                 out_hbm.at[pl.ds(out_base + ci * _GCH, _GCH)],
                                 semo)
                return carry
            lax.fori_loop(0, n_chunks, chunk, 0)
            for _ in range(2 if n_chunks >= 2 else 1):
                pltpu.make_async_copy(
                    rows.at[0], out_hbm.at[pl.ds(out_base, _GCH)], semo).wait()

        run(src_hbm, osrc_hbm)
        run(dst_hbm, odst_hbm)

    return gather_kernel(h_pad, src2d, dst2d)


_ST = 64
_SR = 8
_SCH = _ST * _SR
_HALF = 50000
_TRASH = 2048
_AGGR = _HALF + _TRASH


def _sc_scatter(messages, dst64, n_nodes, n_edges):
    n_chunks = n_edges // _SCH
    k_max = (n_chunks + _NS - 1) // _NS
    zb = 2000
    n_zchunks = _AGGR // zb
    n_ochunks = _HALF // zb
    mesh = plsc.VectorSubcoreMesh(core_axis_name="c", subcore_axis_name="s")

    @functools.partial(
        pl.kernel,
        mesh=mesh,
        compiler_params=pltpu.CompilerParams(use_tc_tiling_on_sc=False),
        out_type=jax.ShapeDtypeStruct((_NC, _HALF, _PAD), jnp.float32),
        scratch_types=[
            pltpu.VMEM((2, _SR, _ST), jnp.int32),
            pltpu.VMEM((2, _SR, _ST), jnp.int32),
            pltpu.VMEM((2, _SCH, _PAD), jnp.float32),
            pltpu.VMEM((zb, _PAD), jnp.float32),
            pltpu.VMEM_SHARED((_AGGR, _PAD), jnp.float32),
            pltpu.SemaphoreType.DMA,
            pltpu.SemaphoreType.DMA,
            pltpu.SemaphoreType.DMA,
            pltpu.SemaphoreType.DMA,
        ],
    )
    def scatter_kernel(msg_hbm, dst_hbm, out_hbm, idxv, idxw, msgb, obuf,
                       aggsh, semi, semm, sema, semo):
        cid = lax.axis_index("c")
        sid = lax.axis_index("s")
        lo = cid * _HALF

        def zrow(i, carry):
            obuf[i, :] = jnp.zeros((_PAD,), jnp.float32)
            return carry
        lax.fori_loop(0, zb, zrow, 0)

        def zchunk(k, carry):
            c = sid + _NS * k

            @pl.when(c < n_zchunks)
            def _():
                pltpu.sync_copy(obuf, aggsh.at[pl.ds(c * zb, zb)])
            return carry
        lax.fori_loop(0, (n_zchunks + _NS - 1) // _NS, zchunk, 0)
        plsc.subcore_barrier()

        pltpu.async_copy(dst_hbm.at[pl.ds(sid * _SR, _SR)], idxv.at[0], semi)
        pltpu.async_copy(msg_hbm.at[pl.ds(sid * _SCH, _SCH)], msgb.at[0],
                         semm)

        def chunk(k, carry):
            c = sid + _NS * k
            b = lax.rem(k, 2)

            @pl.when(c < n_chunks)
            def _():
                pltpu.make_async_copy(dst_hbm.at[pl.ds(sid * _SR, _SR)],
                                      idxv.at[b], semi).wait()
                pltpu.make_async_copy(msg_hbm.at[pl.ds(sid * _SCH, _SCH)],
                                      msgb.at[b], semm).wait()

                @pl.when(k >= 1)
                def _():
                    for j in range(_SR):
                        pltpu.make_async_copy(
                            msgb.at[1 - b, pl.ds(j * _ST, _ST)],
                            aggsh.at[idxw.at[1 - b, 0]], sema).wait()

                cn = c + _NS

                @pl.when(cn < n_chunks)
                def _():
                    pltpu.async_copy(dst_hbm.at[pl.ds(cn * _SR, _SR)],
                                     idxv.at[1 - b], semi)
                    pltpu.async_copy(msg_hbm.at[pl.ds(cn * _SCH, _SCH)],
                                     msgb.at[1 - b], semm)

                for j in range(_SR):
                    for t in range(_ST // 16):
                        v = idxv[b, j, pl.ds(t * 16, 16)]
                        local = v - lo
                        ok = (local >= 0) & (local < _HALF)
                        idxw[b, j, pl.ds(t * 16, 16)] = jnp.where(
                            ok, local, _HALF + (v & (_TRASH - 1)))

                for j in range(_SR):
                    pltpu.async_copy(msgb.at[b, pl.ds(j * _ST, _ST)],
                                     aggsh.at[idxw.at[b, j]], sema, add=True)
            return carry
        lax.fori_loop(0, k_max, chunk, 0)
        for j in range(_SR):
            pltpu.make_async_copy(msgb.at[0, pl.ds(j * _ST, _ST)],
                                  aggsh.at[idxw.at[0, 0]], sema).wait()
        plsc.subcore_barrier()

        def ochunk(k, carry):
            c = sid + _NS * k

            @pl.when(c < n_ochunks)
            def _():
                pltpu.sync_copy(aggsh.at[pl.ds(c * zb, zb)], obuf)
                pltpu.sync_copy(obuf, out_hbm.at[cid].at[pl.ds(c * zb, zb)])
            return carry
        lax.fori_loop(0, (n_ochunks + _NS - 1) // _NS, ochunk, 0)

    return scatter_kernel(messages, dst64)


def _mlp_body(xs_ref, xd_ref, w1s_ref, w1d_ref, b1_ref, w2_ref, b2_ref,
              w3_ref, b3_ref, out_ref):
    dn = (((1,), (1,)), ((), ()))
    m = jnp.maximum(
        jax.lax.dot_general(xs_ref[...], w1s_ref[...], dn,
                            preferred_element_type=jnp.float32)
        + jax.lax.dot_general(xd_ref[...], w1d_ref[...], dn,
                              preferred_element_type=jnp.float32)
        + b1_ref[...], 0.0)
    m = jnp.maximum(
        jax.lax.dot_general(m, w2_ref[...], dn,
                            preferred_element_type=jnp.float32) + b2_ref[...],
        0.0)
    out_ref[...] = jax.lax.dot_general(
        m, w3_ref[...], dn, preferred_element_type=jnp.float32) + b3_ref[...]


def _edge_mlp(xs, xd, w1s, w1d, b1, w2, b2, w3p, b3p):
    n_edges = xs.shape[0]
    grid = n_edges // _EB
    return pl.pallas_call(
        _mlp_body,
        grid=(grid,),
        in_specs=[
            pl.BlockSpec((_EB, _PAD), lambda i: (i, 0)),
            pl.BlockSpec((_EB, _PAD), lambda i: (i, 0)),
            pl.BlockSpec((_H1, _PAD), lambda i: (0, 0)),
            pl.BlockSpec((_H1, _PAD), lambda i: (0, 0)),
            pl.BlockSpec((1, _H1), lambda i: (0, 0)),
            pl.BlockSpec((_H1, _H1), lambda i: (0, 0)),
            pl.BlockSpec((1, _H1), lambda i: (0, 0)),
            pl.BlockSpec((_PAD, _H1), lambda i: (0, 0)),
            pl.BlockSpec((1, _PAD), lambda i: (0, 0)),
        ],
        out_specs=pl.BlockSpec((_EB, _PAD), lambda i: (i, 0)),
        out_shape=jax.ShapeDtypeStruct((n_edges, _PAD), jnp.float32),
        interpret=_INTERPRET,
    )(xs, xd, w1s, w1d, b1, w2, b2, w3p, b3p)


def _gru_body(ni_ref, agg_ref, h_ref, wia_ref, wib_ref, bih_ref,
              whh_ref, bhh_ref, wo_ref, bo_ref, hout_ref, out_ref):
    dn = (((1,), (1,)), ((), ()))
    ni = ni_ref[...]
    agg = agg_ref[...]
    h = h_ref[...][:, :_NF]
    gi = (jax.lax.dot_general(ni, wia_ref[...], dn,
                              preferred_element_type=jnp.float32)
          + jax.lax.dot_general(agg, wib_ref[...], dn,
                                preferred_element_type=jnp.float32)
          + bih_ref[...])
    gh = jax.lax.dot_general(h, whh_ref[...], dn,
                             preferred_element_type=jnp.float32) + bhh_ref[...]
    i_r = gi[:, :_NF]
    i_z = gi[:, _NF:2 * _NF]
    i_n = gi[:, 2 * _NF:]
    h_r = gh[:, :_NF]
    h_z = gh[:, _NF:2 * _NF]
    h_n = gh[:, 2 * _NF:]
    r = jax.nn.sigmoid(i_r + h_r)
    z = jax.nn.sigmoid(i_z + h_z)
    n = jnp.tanh(i_n + r * h_n)
    hn = (1.0 - z) * n + z * h
    hout_ref[...] = jnp.pad(hn, ((0, 0), (0, _PAD - _NF)))
    out_ref[...] = jax.lax.dot_general(
        hn, wo_ref[...], dn, preferred_element_type=jnp.float32) + bo_ref[...]


def _gru_step(node_inputs, aggp, h_pad, wia, wib, b_ih, w_hh, b_hh, wo, bo):
    n_nodes = node_inputs.shape[0]
    grid = n_nodes // _NB
    return pl.pallas_call(
        _gru_body,
        grid=(grid,),
        in_specs=[
            pl.BlockSpec((_NB, _NI), lambda i: (i, 0)),
            pl.BlockSpec((_NB, _PAD), lambda i: (i, 0)),
            pl.BlockSpec((_NB, _PAD), lambda i: (i, 0)),
            pl.BlockSpec((3 * _NF, _NI), lambda i: (0, 0)),
            pl.BlockSpec((3 * _NF, _PAD), lambda i: (0, 0)),
            pl.BlockSpec((1, 3 * _NF), lambda i: (0, 0)),
            pl.BlockSpec((3 * _NF, _NF), lambda i: (0, 0)),
            pl.BlockSpec((1, 3 * _NF), lambda i: (0, 0)),
            pl.BlockSpec((_NO, _NF), lambda i: (0, 0)),
            pl.BlockSpec((1, _NO), lambda i: (0, 0)),
        ],
        out_specs=[
            pl.BlockSpec((_NB, _PAD), lambda i: (i, 0)),
            pl.BlockSpec((_NB, _NO), lambda i: (i, 0)),
        ],
        out_shape=[
            jax.ShapeDtypeStruct((n_nodes, _PAD), jnp.float32),
            jax.ShapeDtypeStruct((n_nodes, _NO), jnp.float32),
        ],
        interpret=_INTERPRET,
    )(node_inputs, aggp, h_pad, wia, wib, b_ih, w_hh, b_hh, wo, bo)


def kernel(node_inputs, src_ids, dst_ids, W1, b1, W2, b2, W3, b3,
           W_ih, b_ih, W_hh, b_hh, Wo, bo):
    n_nodes = node_inputs.shape[0]
    n_edges = src_ids.shape[0]

    src2d = src_ids.reshape(n_edges // _GT, _GT)
    dst2d = dst_ids.reshape(n_edges // _GT, _GT)
    dst64 = dst_ids.reshape(n_edges // _ST, _ST)

    w1s = jnp.zeros((_H1, _PAD), jnp.float32).at[:, :_NF].set(W1[:, :_NF])
    w1d = jnp.zeros((_H1, _PAD), jnp.float32).at[:, :_NF].set(W1[:, _NF:])
    w3p = jnp.zeros((_PAD, _H1), jnp.float32).at[:_EF, :].set(W3)
    b3p = jnp.zeros((1, _PAD), jnp.float32).at[0, :_EF].set(b3)
    wia = W_ih[:, :_NI]
    wib = jnp.zeros((3 * _NF, _PAD), jnp.float32).at[:, :_EF].set(W_ih[:, _NI:])

    b1r = b1.reshape(1, -1)
    b2r = b2.reshape(1, -1)
    bihr = b_ih.reshape(1, -1)
    bhhr = b_hh.reshape(1, -1)
    bor = bo.reshape(1, -1)

    h_pad = jnp.zeros((n_nodes, _PAD), jnp.float32)
    outputs = []
    for _ in range(_N_ITERS):
        xs, xd = _sc_gather(h_pad, src2d, dst2d, n_edges)
        messages = _edge_mlp(xs, xd, w1s, w1d, b1r, W2, b2r, w3p, b3p)
        aggp = _sc_scatter(messages, dst64, n_nodes, n_edges)
        agg = aggp.reshape(n_nodes, _PAD)
        h_pad, out_t = _gru_step(node_inputs, agg, h_pad, wia, wib, bihr,
                                 W_hh, bhhr, Wo, bor)
        outputs.append(out_t)
    return jnp.stack(outputs, axis=0)


# speedup vs baseline: 14.4514x; 2.4685x over previous
import functools
import jax
import jax.numpy as jnp
from jax import lax
from jax.experimental import pallas as pl
from jax.experimental.pallas import tpu as pltpu
from jax.experimental.pallas import tpu_sc as plsc

_N_ITERS = 7
_NF = 10
_NI = 9
_EF = 11
_NO = 9
_H1 = 96
_PAD = 16

_EB = 12800
_NB = 2000

_NC = 2
_NS = 16
_NW = _NC * _NS
_GT = 125
_GK = 8
_GCH = _GT * _GK

_INTERPRET = False


def _sc_gather(h_pad, src2d, dst2d, n_edges):
    per_w = n_edges // _NW
    n_chunks = per_w // _GCH
    rows_per_chunk = _GK
    mesh = plsc.VectorSubcoreMesh(core_axis_name="c", subcore_axis_name="s")

    @functools.partial(
        pl.kernel,
        mesh=mesh,
        compiler_params=pltpu.CompilerParams(use_tc_tiling_on_sc=False),
        out_type=[
            jax.ShapeDtypeStruct((n_edges, _PAD), jnp.float32),
            jax.ShapeDtypeStruct((n_edges, _PAD), jnp.float32),
        ],
        scratch_types=[
            pltpu.VMEM((2, _GK, _GT), jnp.int32),
            pltpu.VMEM((2, _GCH, _PAD), jnp.float32),
            pltpu.SemaphoreType.DMA,
            pltpu.SemaphoreType.DMA,
            pltpu.SemaphoreType.DMA,
        ],
    )
    def gather_kernel(h_hbm, src_hbm, dst_hbm, osrc_hbm, odst_hbm,
                      idxv, rows, semi, semg, semo):
        wid = lax.axis_index("s") * _NC + lax.axis_index("c")
        row_base = wid * (per_w // _GT)
        out_base = wid * per_w

        def run(idx_hbm, out_hbm):
            pltpu.async_copy(idx_hbm.at[pl.ds(row_base, rows_per_chunk)],
                             idxv.at[0], semi)

            def chunk(ci, carry):
                b = lax.rem(ci, 2)
                pltpu.make_async_copy(
                    idx_hbm.at[pl.ds(row_base, rows_per_chunk)],
                    idxv.at[b], semi).wait()

                @pl.when(ci + 1 < n_chunks)
                def _():
                    pltpu.async_copy(
                        idx_hbm.at[pl.ds(row_base + (ci + 1) * rows_per_chunk,
                                         rows_per_chunk)],
                        idxv.at[1 - b], semi)

                @pl.when(ci >= 2)
                def _():
                    pltpu.make_async_copy(
                        rows.at[b],
                        out_hbm.at[pl.ds(out_base, _GCH)], semo).wait()

                cps = [
                    pltpu.async_copy(h_hbm.at[idxv.at[b, j]],
                                     rows.at[b, pl.ds(j * _GT, _GT)], semg)
                    for j in range(_GK)
                ]
                for c in cps:
                    c.wait()
                pltpu.async_copy(rows.at[b],
                                 out_hbm.at[pl.ds(out_base + ci * _GCH, _GCH)],
                                 semo)
                return carry
            lax.fori_loop(0, n_chunks, chunk, 0)
            for _ in range(2 if n_chunks >= 2 else 1):
                pltpu.make_async_copy(
                    rows.at[0], out_hbm.at[pl.ds(out_base, _GCH)], semo).wait()

        run(src_hbm, osrc_hbm)
        run(dst_hbm, odst_hbm)

    return gather_kernel(h_pad, src2d, dst2d)


_ST = 64
_SR = 8
_SCH = _ST * _SR
_HALF = 50000
_TRASH = 2048
_AGGR = _HALF + _TRASH


def _sc_scatter(messages, dst64, n_nodes, n_edges):
    n_chunks = n_edges // _SCH
    k_max = (n_chunks + _NS - 1) // _NS
    zb = 2000
    n_zchunks = _AGGR // zb
    n_ochunks = _HALF // zb
    mesh = plsc.VectorSubcoreMesh(core_axis_name="c", subcore_axis_name="s")

    @functools.partial(
        pl.kernel,
        mesh=mesh,
        compiler_params=pltpu.CompilerParams(use_tc_tiling_on_sc=False),
        out_type=jax.ShapeDtypeStruct((_NC, _HALF, _PAD), jnp.float32),
        scratch_types=[
            pltpu.VMEM((2, _SR, _ST), jnp.int32),
            pltpu.VMEM((2, _SR, _ST), jnp.int32),
            pltpu.VMEM((2, _SCH, _PAD), jnp.float32),
            pltpu.VMEM((zb, _PAD), jnp.float32),
            pltpu.VMEM_SHARED((_AGGR, _PAD), jnp.float32),
            pltpu.SemaphoreType.DMA,
            pltpu.SemaphoreType.DMA,
            pltpu.SemaphoreType.DMA,
            pltpu.SemaphoreType.DMA,
        ],
    )
    def scatter_kernel(msg_hbm, dst_hbm, out_hbm, idxv, idxw, msgb, obuf,
                       aggsh, semi, semm, sema, semo):
        cid = lax.axis_index("c")
        sid = lax.axis_index("s")
        lo = cid * _HALF

        def zrow(i, carry):
            obuf[i, :] = jnp.zeros((_PAD,), jnp.float32)
            return carry
        lax.fori_loop(0, zb, zrow, 0)

        def zchunk(k, carry):
            c = sid + _NS * k

            @pl.when(c < n_zchunks)
            def _():
                pltpu.sync_copy(obuf, aggsh.at[pl.ds(c * zb, zb)])
            return carry
        lax.fori_loop(0, (n_zchunks + _NS - 1) // _NS, zchunk, 0)
        plsc.subcore_barrier()

        pltpu.async_copy(dst_hbm.at[pl.ds(sid * _SR, _SR)], idxv.at[0], semi)
        pltpu.async_copy(msg_hbm.at[pl.ds(sid * _SCH, _SCH)], msgb.at[0],
                         semm)

        def chunk(k, carry):
            c = sid + _NS * k
            b = lax.rem(k, 2)

            @pl.when(c < n_chunks)
            def _():
                pltpu.make_async_copy(dst_hbm.at[pl.ds(sid * _SR, _SR)],
                                      idxv.at[b], semi).wait()
                pltpu.make_async_copy(msg_hbm.at[pl.ds(sid * _SCH, _SCH)],
                                      msgb.at[b], semm).wait()

                @pl.when(k >= 1)
                def _():
                    for j in range(_SR):
                        pltpu.make_async_copy(
                            msgb.at[1 - b, pl.ds(j * _ST, _ST)],
                            aggsh.at[idxw.at[1 - b, 0]], sema).wait()

                cn = c + _NS

                @pl.when(cn < n_chunks)
                def _():
                    pltpu.async_copy(dst_hbm.at[pl.ds(cn * _SR, _SR)],
                                     idxv.at[1 - b], semi)
                    pltpu.async_copy(msg_hbm.at[pl.ds(cn * _SCH, _SCH)],
                                     msgb.at[1 - b], semm)

                for j in range(_SR):
                    for t in range(_ST // 16):
                        v = idxv[b, j, pl.ds(t * 16, 16)]
                        local = v - lo
                        ok = (local >= 0) & (local < _HALF)
                        idxw[b, j, pl.ds(t * 16, 16)] = jnp.where(
                            ok, local, _HALF + (v & (_TRASH - 1)))

                for j in range(_SR):
                    pltpu.async_copy(msgb.at[b, pl.ds(j * _ST, _ST)],
                                     aggsh.at[idxw.at[b, j]], sema, add=True)
            return carry
        lax.fori_loop(0, k_max, chunk, 0)
        for j in range(_SR):
            pltpu.make_async_copy(msgb.at[0, pl.ds(j * _ST, _ST)],
                                  aggsh.at[idxw.at[0, 0]], sema).wait()
        plsc.subcore_barrier()

        def ochunk(k, carry):
            c = sid + _NS * k

            @pl.when(c < n_ochunks)
            def _():
                pltpu.sync_copy(aggsh.at[pl.ds(c * zb, zb)], obuf)
                pltpu.sync_copy(obuf, out_hbm.at[cid].at[pl.ds(c * zb, zb)])
            return carry
        lax.fori_loop(0, (n_ochunks + _NS - 1) // _NS, ochunk, 0)

    return scatter_kernel(messages, dst64)


_EBP = 1600


def _mlp_body(xs_ref, xd_ref, w1s_ref, w1d_ref, b1_ref, w2_ref, b2_ref,
              w3_ref, b3_ref, out_ref):
    dn = (((1,), (1,)), ((), ()))
    m = jnp.maximum(
        jax.lax.dot_general(xs_ref[...], w1s_ref[...], dn,
                            preferred_element_type=jnp.float32)
        + jax.lax.dot_general(xd_ref[...], w1d_ref[...], dn,
                              preferred_element_type=jnp.float32)
        + b1_ref[...], 0.0)
    m = jnp.maximum(
        jax.lax.dot_general(m, w2_ref[...], dn,
                            preferred_element_type=jnp.float32) + b2_ref[...],
        0.0)
    out_ref[...] = jax.lax.dot_general(
        m, w3_ref[...], dn, preferred_element_type=jnp.float32) + b3_ref[...]


def _edge_mlp(xs, xd, w1s_bd, w1d_bd, b1t, w2_bd, b2t, w3_bd, b3t):
    n_rows = xs.shape[0]
    grid = n_rows // _EBP
    h8 = 8 * _H1
    return pl.pallas_call(
        _mlp_body,
        grid=(grid,),
        in_specs=[
            pl.BlockSpec((_EBP, 128), lambda i: (i, 0)),
            pl.BlockSpec((_EBP, 128), lambda i: (i, 0)),
            pl.BlockSpec((h8, 128), lambda i: (0, 0)),
            pl.BlockSpec((h8, 128), lambda i: (0, 0)),
            pl.BlockSpec((1, h8), lambda i: (0, 0)),
            pl.BlockSpec((h8, h8), lambda i: (0, 0)),
            pl.BlockSpec((1, h8), lambda i: (0, 0)),
            pl.BlockSpec((128, h8), lambda i: (0, 0)),
            pl.BlockSpec((1, 128), lambda i: (0, 0)),
        ],
        out_specs=pl.BlockSpec((_EBP, 128), lambda i: (i, 0)),
        out_shape=jax.ShapeDtypeStruct((n_rows, 128), jnp.float32),
        interpret=_INTERPRET,
    )(xs, xd, w1s_bd, w1d_bd, b1t, w2_bd, b2t, w3_bd, b3t)


def _gru_body(ni_ref, agg_ref, h_ref, wia_ref, wib_ref, bih_ref,
              whh_ref, bhh_ref, wo_ref, bo_ref, hout_ref, out_ref):
    dn = (((1,), (1,)), ((), ()))
    ni = ni_ref[...]
    agg = agg_ref[...]
    h = h_ref[...][:, :_NF]
    gi = (jax.lax.dot_general(ni, wia_ref[...], dn,
                              preferred_element_type=jnp.float32)
          + jax.lax.dot_general(agg, wib_ref[...], dn,
                                preferred_element_type=jnp.float32)
          + bih_ref[...])
    gh = jax.lax.dot_general(h, whh_ref[...], dn,
                             preferred_element_type=jnp.float32) + bhh_ref[...]
    i_r = gi[:, :_NF]
    i_z = gi[:, _NF:2 * _NF]
    i_n = gi[:, 2 * _NF:]
    h_r = gh[:, :_NF]
    h_z = gh[:, _NF:2 * _NF]
    h_n = gh[:, 2 * _NF:]
    r = jax.nn.sigmoid(i_r + h_r)
    z = jax.nn.sigmoid(i_z + h_z)
    n = jnp.tanh(i_n + r * h_n)
    hn = (1.0 - z) * n + z * h
    hout_ref[...] = jnp.pad(hn, ((0, 0), (0, _PAD - _NF)))
    out_ref[...] = jax.lax.dot_general(
        hn, wo_ref[...], dn, preferred_element_type=jnp.float32) + bo_ref[...]


def _gru_step(node_inputs, aggp, h_pad, wia, wib, b_ih, w_hh, b_hh, wo, bo):
    n_nodes = node_inputs.shape[0]
    grid = n_nodes // _NB
    return pl.pallas_call(
        _gru_body,
        grid=(grid,),
        in_specs=[
            pl.BlockSpec((_NB, _NI), lambda i: (i, 0)),
            pl.BlockSpec((_NB, _PAD), lambda i: (i, 0)),
            pl.BlockSpec((_NB, _PAD), lambda i: (i, 0)),
            pl.BlockSpec((3 * _NF, _NI), lambda i: (0, 0)),
            pl.BlockSpec((3 * _NF, _PAD), lambda i: (0, 0)),
            pl.BlockSpec((1, 3 * _NF), lambda i: (0, 0)),
            pl.BlockSpec((3 * _NF, _NF), lambda i: (0, 0)),
            pl.BlockSpec((1, 3 * _NF), lambda i: (0, 0)),
            pl.BlockSpec((_NO, _NF), lambda i: (0, 0)),
            pl.BlockSpec((1, _NO), lambda i: (0, 0)),
        ],
        out_specs=[
            pl.BlockSpec((_NB, _PAD), lambda i: (i, 0)),
            pl.BlockSpec((_NB, _NO), lambda i: (i, 0)),
        ],
        out_shape=[
            jax.ShapeDtypeStruct((n_nodes, _PAD), jnp.float32),
            jax.ShapeDtypeStruct((n_nodes, _NO), jnp.float32),
        ],
        interpret=_INTERPRET,
    )(node_inputs, aggp, h_pad, wia, wib, b_ih, w_hh, b_hh, wo, bo)


def kernel(node_inputs, src_ids, dst_ids, W1, b1, W2, b2, W3, b3,
           W_ih, b_ih, W_hh, b_hh, Wo, bo):
    n_nodes = node_inputs.shape[0]
    n_edges = src_ids.shape[0]

    src2d = src_ids.reshape(n_edges // _GT, _GT)
    dst2d = dst_ids.reshape(n_edges // _GT, _GT)
    dst64 = dst_ids.reshape(n_edges // _ST, _ST)

    w1s = jnp.zeros((_H1, _PAD), jnp.float32).at[:, :_NF].set(W1[:, :_NF])
    w1d = jnp.zeros((_H1, _PAD), jnp.float32).at[:, :_NF].set(W1[:, _NF:])
    w3p = jnp.zeros((_PAD, _H1), jnp.float32).at[:_EF, :].set(W3)
    b3p = jnp.zeros((_PAD,), jnp.float32).at[:_EF].set(b3)
    wia = W_ih[:, :_NI]
    wib = jnp.zeros((3 * _NF, _PAD), jnp.float32).at[:, :_EF].set(W_ih[:, _NI:])

    eye8 = jnp.eye(8, dtype=jnp.float32)
    w1s_bd = jnp.einsum('ab,ij->aibj', eye8, w1s).reshape(8 * _H1, 128)
    w1d_bd = jnp.einsum('ab,ij->aibj', eye8, w1d).reshape(8 * _H1, 128)
    w2_bd = jnp.einsum('ab,ij->aibj', eye8, W2).reshape(8 * _H1, 8 * _H1)
    w3_bd = jnp.einsum('ab,ij->aibj', eye8, w3p).reshape(128, 8 * _H1)
    b1t = jnp.tile(b1, 8).reshape(1, -1)
    b2t = jnp.tile(b2, 8).reshape(1, -1)
    b3t = jnp.tile(b3p, 8).reshape(1, -1)

    b1r = b1.reshape(1, -1)
    b2r = b2.reshape(1, -1)
    bihr = b_ih.reshape(1, -1)
    bhhr = b_hh.reshape(1, -1)
    bor = bo.reshape(1, -1)

    h_pad = jnp.zeros((n_nodes, _PAD), jnp.float32)
    outputs = []
    for _ in range(_N_ITERS):
        xs, xd = _sc_gather(h_pad, src2d, dst2d, n_edges)
        xs8 = xs.reshape(n_edges // 8, 128)
        xd8 = xd.reshape(n_edges // 8, 128)
        msg8 = _edge_mlp(xs8, xd8, w1s_bd, w1d_bd, b1t, w2_bd, b2t, w3_bd,
                         b3t)
        messages = msg8.reshape(n_edges, _PAD)
        aggp = _sc_scatter(messages, dst64, n_nodes, n_edges)
        agg = aggp.reshape(n_nodes, _PAD)
        h_pad, out_t = _gru_step(node_inputs, agg, h_pad, wia, wib, bihr,
                                 W_hh, bhhr, Wo, bor)
        outputs.append(out_t)
    return jnp.stack(outputs, axis=0)


# speedup vs baseline: 16.4791x; 1.1403x over previous
import functools
import jax
import jax.numpy as jnp
from jax import lax
from jax.experimental import pallas as pl
from jax.experimental.pallas import tpu as pltpu
from jax.experimental.pallas import tpu_sc as plsc

_N_ITERS = 7
_NF = 10
_NI = 9
_EF = 11
_NO = 9
_H1 = 96
_PAD = 16

_EB = 12800
_NB = 2000

_NC = 2
_NS = 16
_NW = _NC * _NS
_GT = 125
_GK = 8
_GCH = _GT * _GK

_INTERPRET = False


def _sc_gather(h_pad, src2d, dst2d, n_edges):
    per_w = n_edges // _NW
    n_chunks = per_w // _GCH
    rows_per_chunk = _GK
    mesh = plsc.VectorSubcoreMesh(core_axis_name="c", subcore_axis_name="s")

    @functools.partial(
        pl.kernel,
        mesh=mesh,
        compiler_params=pltpu.CompilerParams(use_tc_tiling_on_sc=False),
        out_type=[
            jax.ShapeDtypeStruct((n_edges, _PAD), jnp.float32),
            jax.ShapeDtypeStruct((n_edges, _PAD), jnp.float32),
        ],
        scratch_types=[
            pltpu.VMEM((2, _GK, _GT), jnp.int32),
            pltpu.VMEM((2, _GCH, _PAD), jnp.float32),
            pltpu.SemaphoreType.DMA,
            pltpu.SemaphoreType.DMA,
            pltpu.SemaphoreType.DMA,
        ],
    )
    def gather_kernel(h_hbm, src_hbm, dst_hbm, osrc_hbm, odst_hbm,
                      idxv, rows, semi, semg, semo):
        wid = lax.axis_index("s") * _NC + lax.axis_index("c")
        row_base = wid * (per_w // _GT)
        out_base = wid * per_w

        def run(idx_hbm, out_hbm):
            pltpu.async_copy(idx_hbm.at[pl.ds(row_base, rows_per_chunk)],
                             idxv.at[0], semi)

            def chunk(ci, carry):
                b = lax.rem(ci, 2)
                pltpu.make_async_copy(
                    idx_hbm.at[pl.ds(row_base, rows_per_chunk)],
                    idxv.at[b], semi).wait()

                @pl.when(ci + 1 < n_chunks)
                def _():
                    pltpu.async_copy(
                        idx_hbm.at[pl.ds(row_base + (ci + 1) * rows_per_chunk,
                                         rows_per_chunk)],
                        idxv.at[1 - b], semi)

                @pl.when(ci >= 2)
                def _():
                    pltpu.make_async_copy(
                        rows.at[b],
                        out_hbm.at[pl.ds(out_base, _GCH)], semo).wait()

                cps = [
                    pltpu.async_copy(h_hbm.at[idxv.at[b, j]],
                                     rows.at[b, pl.ds(j * _GT, _GT)], semg)
                    for j in range(_GK)
                ]
                for c in cps:
                    c.wait()
                pltpu.async_copy(rows.at[b],
                                 out_hbm.at[pl.ds(out_base + ci * _GCH, _GCH)],
                                 semo)
                return carry
            lax.fori_loop(0, n_chunks, chunk, 0)
            for _ in range(2 if n_chunks >= 2 else 1):
                pltpu.make_async_copy(
                    rows.at[0], out_hbm.at[pl.ds(out_base, _GCH)], semo).wait()

        run(src_hbm, osrc_hbm)
        run(dst_hbm, odst_hbm)

    return gather_kernel(h_pad, src2d, dst2d)


_ST = 64
_SR = 8
_SCH = _ST * _SR
_HALF = 50000
_TRASH = 2048
_AGGR = _HALF + _TRASH


def _sc_scatter(messages, dst64, n_nodes, n_edges, msg0=None):
    const_mode = messages is None
    n_chunks = n_edges // _SCH
    k_max = (n_chunks + _NS - 1) // _NS
    zb = 2000
    n_zchunks = _AGGR // zb
    n_ochunks = _HALF // zb
    mesh = plsc.VectorSubcoreMesh(core_axis_name="c", subcore_axis_name="s")

    @functools.partial(
        pl.kernel,
        mesh=mesh,
        compiler_params=pltpu.CompilerParams(use_tc_tiling_on_sc=False),
        out_type=jax.ShapeDtypeStruct((_NC, _HALF, _PAD), jnp.float32),
        scratch_types=[
            pltpu.VMEM((2, _SR, _ST), jnp.int32),
            pltpu.VMEM((2, _SR, _ST), jnp.int32),
            pltpu.VMEM((2, _SCH, _PAD), jnp.float32),
            pltpu.VMEM((zb, _PAD), jnp.float32),
            pltpu.VMEM_SHARED((_AGGR, _PAD), jnp.float32),
            pltpu.SemaphoreType.DMA,
            pltpu.SemaphoreType.DMA,
            pltpu.SemaphoreType.DMA,
            pltpu.SemaphoreType.DMA,
        ],
    )
    def scatter_kernel(msg_hbm, dst_hbm, out_hbm, idxv, idxw, msgb, obuf,
                       aggsh, semi, semm, sema, semo):
        cid = lax.axis_index("c")
        sid = lax.axis_index("s")
        lo = cid * _HALF

        def zrow(i, carry):
            obuf[i, :] = jnp.zeros((_PAD,), jnp.float32)
            return carry
        lax.fori_loop(0, zb, zrow, 0)

        def zchunk(k, carry):
            c = sid + _NS * k

            @pl.when(c < n_zchunks)
            def _():
                pltpu.sync_copy(obuf, aggsh.at[pl.ds(c * zb, zb)])
            return carry
        lax.fori_loop(0, (n_zchunks + _NS - 1) // _NS, zchunk, 0)
        plsc.subcore_barrier()

        if const_mode:
            pltpu.sync_copy(msg_hbm, msgb.at[0, pl.ds(0, 1)])
            rowf = msgb[0, 0, :]

            def frow(i, carry):
                msgb[0, i, :] = rowf
                msgb[1, i, :] = rowf
                return carry
            lax.fori_loop(0, _SCH, frow, 0)

        pltpu.async_copy(dst_hbm.at[pl.ds(sid * _SR, _SR)], idxv.at[0], semi)
        if not const_mode:
            pltpu.async_copy(msg_hbm.at[pl.ds(sid * _SCH, _SCH)], msgb.at[0],
                             semm)

        def chunk(k, carry):
            c = sid + _NS * k
            b = lax.rem(k, 2)

            @pl.when(c < n_chunks)
            def _():
                pltpu.make_async_copy(dst_hbm.at[pl.ds(sid * _SR, _SR)],
                                      idxv.at[b], semi).wait()
                if not const_mode:
                    pltpu.make_async_copy(
                        msg_hbm.at[pl.ds(sid * _SCH, _SCH)],
                        msgb.at[b], semm).wait()

                @pl.when(k >= 1)
                def _():
                    for j in range(_SR):
                        pltpu.make_async_copy(
                            msgb.at[1 - b, pl.ds(j * _ST, _ST)],
                            aggsh.at[idxw.at[1 - b, 0]], sema).wait()

                cn = c + _NS

                @pl.when(cn < n_chunks)
                def _():
                    pltpu.async_copy(dst_hbm.at[pl.ds(cn * _SR, _SR)],
                                     idxv.at[1 - b], semi)
                    if not const_mode:
                        pltpu.async_copy(msg_hbm.at[pl.ds(cn * _SCH, _SCH)],
                                         msgb.at[1 - b], semm)

                for j in range(_SR):
                    for t in range(_ST // 16):
                        v = idxv[b, j, pl.ds(t * 16, 16)]
                        local = v - lo
                        ok = (local >= 0) & (local < _HALF)
                        idxw[b, j, pl.ds(t * 16, 16)] = jnp.where(
                            ok, local, _HALF + (v & (_TRASH - 1)))

                for j in range(_SR):
                    pltpu.async_copy(msgb.at[b, pl.ds(j * _ST, _ST)],
                                     aggsh.at[idxw.at[b, j]], sema, add=True)
            return carry
        lax.fori_loop(0, k_max, chunk, 0)
        for j in range(_SR):
            pltpu.make_async_copy(msgb.at[0, pl.ds(j * _ST, _ST)],
                                  aggsh.at[idxw.at[0, 0]], sema).wait()
        plsc.subcore_barrier()

        def ochunk(k, carry):
            c = sid + _NS * k

            @pl.when(c < n_ochunks)
            def _():
                pltpu.sync_copy(aggsh.at[pl.ds(c * zb, zb)], obuf)
                pltpu.sync_copy(obuf, out_hbm.at[cid].at[pl.ds(c * zb, zb)])
            return carry
        lax.fori_loop(0, (n_ochunks + _NS - 1) // _NS, ochunk, 0)

    if const_mode:
        return scatter_kernel(msg0, dst64)
    return scatter_kernel(messages, dst64)


_EBP = 4000


def _mlp_body(xs_ref, xd_ref, w1s_ref, w1d_ref, b1_ref, w2_ref, b2_ref,
              w3_ref, b3_ref, out_ref):
    dn = (((1,), (1,)), ((), ()))
    m = jnp.maximum(
        jax.lax.dot_general(xs_ref[...], w1s_ref[...], dn,
                            preferred_element_type=jnp.float32)
        + jax.lax.dot_general(xd_ref[...], w1d_ref[...], dn,
                              preferred_element_type=jnp.float32)
        + b1_ref[...], 0.0)
    m = jnp.maximum(
        jax.lax.dot_general(m, w2_ref[...], dn,
                            preferred_element_type=jnp.float32) + b2_ref[...],
        0.0)
    out_ref[...] = jax.lax.dot_general(
        m, w3_ref[...], dn, preferred_element_type=jnp.float32) + b3_ref[...]


def _edge_mlp(xs, xd, w1s_bd, w1d_bd, b1t, w2_bd, b2t, w3_bd, b3t):
    n_rows = xs.shape[0]
    grid = n_rows // _EBP
    h8 = 8 * _H1
    return pl.pallas_call(
        _mlp_body,
        grid=(grid,),
        in_specs=[
            pl.BlockSpec((_EBP, 128), lambda i: (i, 0)),
            pl.BlockSpec((_EBP, 128), lambda i: (i, 0)),
            pl.BlockSpec((h8, 128), lambda i: (0, 0)),
            pl.BlockSpec((h8, 128), lambda i: (0, 0)),
            pl.BlockSpec((1, h8), lambda i: (0, 0)),
            pl.BlockSpec((h8, h8), lambda i: (0, 0)),
            pl.BlockSpec((1, h8), lambda i: (0, 0)),
            pl.BlockSpec((128, h8), lambda i: (0, 0)),
            pl.BlockSpec((1, 128), lambda i: (0, 0)),
        ],
        out_specs=pl.BlockSpec((_EBP, 128), lambda i: (i, 0)),
        out_shape=jax.ShapeDtypeStruct((n_rows, 128), jnp.float32),
        interpret=_INTERPRET,
    )(xs, xd, w1s_bd, w1d_bd, b1t, w2_bd, b2t, w3_bd, b3t)


def _gru_body(ni_ref, agg_ref, h_ref, wia_ref, wib_ref, bih_ref,
              whh_ref, bhh_ref, wo_ref, bo_ref, hout_ref, out_ref):
    dn = (((1,), (1,)), ((), ()))
    ni = ni_ref[...]
    agg = agg_ref[...]
    h = h_ref[...][:, :_NF]
    gi = (jax.lax.dot_general(ni, wia_ref[...], dn,
                              preferred_element_type=jnp.float32)
          + jax.lax.dot_general(agg, wib_ref[...], dn,
                                preferred_element_type=jnp.float32)
          + bih_ref[...])
    gh = jax.lax.dot_general(h, whh_ref[...], dn,
                             preferred_element_type=jnp.float32) + bhh_ref[...]
    i_r = gi[:, :_NF]
    i_z = gi[:, _NF:2 * _NF]
    i_n = gi[:, 2 * _NF:]
    h_r = gh[:, :_NF]
    h_z = gh[:, _NF:2 * _NF]
    h_n = gh[:, 2 * _NF:]
    r = jax.nn.sigmoid(i_r + h_r)
    z = jax.nn.sigmoid(i_z + h_z)
    n = jnp.tanh(i_n + r * h_n)
    hn = (1.0 - z) * n + z * h
    hout_ref[...] = jnp.pad(hn, ((0, 0), (0, _PAD - _NF)))
    out_ref[...] = jax.lax.dot_general(
        hn, wo_ref[...], dn, preferred_element_type=jnp.float32) + bo_ref[...]


def _gru_step(node_inputs, aggp, h_pad, wia, wib, b_ih, w_hh, b_hh, wo, bo):
    n_nodes = node_inputs.shape[0]
    grid = n_nodes // _NB
    return pl.pallas_call(
        _gru_body,
        grid=(grid,),
        in_specs=[
            pl.BlockSpec((_NB, _NI), lambda i: (i, 0)),
            pl.BlockSpec((_NB, _PAD), lambda i: (i, 0)),
            pl.BlockSpec((_NB, _PAD), lambda i: (i, 0)),
            pl.BlockSpec((3 * _NF, _NI), lambda i: (0, 0)),
            pl.BlockSpec((3 * _NF, _PAD), lambda i: (0, 0)),
            pl.BlockSpec((1, 3 * _NF), lambda i: (0, 0)),
            pl.BlockSpec((3 * _NF, _NF), lambda i: (0, 0)),
            pl.BlockSpec((1, 3 * _NF), lambda i: (0, 0)),
            pl.BlockSpec((_NO, _NF), lambda i: (0, 0)),
            pl.BlockSpec((1, _NO), lambda i: (0, 0)),
        ],
        out_specs=[
            pl.BlockSpec((_NB, _PAD), lambda i: (i, 0)),
            pl.BlockSpec((_NB, _NO), lambda i: (i, 0)),
        ],
        out_shape=[
            jax.ShapeDtypeStruct((n_nodes, _PAD), jnp.float32),
            jax.ShapeDtypeStruct((n_nodes, _NO), jnp.float32),
        ],
        interpret=_INTERPRET,
    )(node_inputs, aggp, h_pad, wia, wib, b_ih, w_hh, b_hh, wo, bo)


def kernel(node_inputs, src_ids, dst_ids, W1, b1, W2, b2, W3, b3,
           W_ih, b_ih, W_hh, b_hh, Wo, bo):
    n_nodes = node_inputs.shape[0]
    n_edges = src_ids.shape[0]

    src2d = src_ids.reshape(n_edges // _GT, _GT)
    dst2d = dst_ids.reshape(n_edges // _GT, _GT)
    dst64 = dst_ids.reshape(n_edges // _ST, _ST)

    w1s = jnp.zeros((_H1, _PAD), jnp.float32).at[:, :_NF].set(W1[:, :_NF])
    w1d = jnp.zeros((_H1, _PAD), jnp.float32).at[:, :_NF].set(W1[:, _NF:])
    w3p = jnp.zeros((_PAD, _H1), jnp.float32).at[:_EF, :].set(W3)
    b3p = jnp.zeros((_PAD,), jnp.float32).at[:_EF].set(b3)
    wia = W_ih[:, :_NI]
    wib = jnp.zeros((3 * _NF, _PAD), jnp.float32).at[:, :_EF].set(W_ih[:, _NI:])

    eye8 = jnp.eye(8, dtype=jnp.float32)
    w1s_bd = jnp.einsum('ab,ij->aibj', eye8, w1s).reshape(8 * _H1, 128)
    w1d_bd = jnp.einsum('ab,ij->aibj', eye8, w1d).reshape(8 * _H1, 128)
    w2_bd = jnp.einsum('ab,ij->aibj', eye8, W2).reshape(8 * _H1, 8 * _H1)
    w3_bd = jnp.einsum('ab,ij->aibj', eye8, w3p).reshape(128, 8 * _H1)
    b1t = jnp.tile(b1, 8).reshape(1, -1)
    b2t = jnp.tile(b2, 8).reshape(1, -1)
    b3t = jnp.tile(b3p, 8).reshape(1, -1)

    b1r = b1.reshape(1, -1)
    b2r = b2.reshape(1, -1)
    bihr = b_ih.reshape(1, -1)
    bhhr = b_hh.reshape(1, -1)
    bor = bo.reshape(1, -1)

    m0 = jnp.maximum(b1, 0.0)
    m0 = jnp.maximum(m0 @ W2.T + b2, 0.0)
    msg0 = (m0 @ w3p.T + b3p).reshape(1, _PAD)

    h_pad = jnp.zeros((n_nodes, _PAD), jnp.float32)
    outputs = []
    for it in range(_N_ITERS):
        if it == 0:
            aggp = _sc_scatter(None, dst64, n_nodes, n_edges, msg0=msg0)
        else:
            xs, xd = _sc_gather(h_pad, src2d, dst2d, n_edges)
            xs8 = xs.reshape(n_edges // 8, 128)
            xd8 = xd.reshape(n_edges // 8, 128)
            msg8 = _edge_mlp(xs8, xd8, w1s_bd, w1d_bd, b1t, w2_bd, b2t,
                             w3_bd, b3t)
            messages = msg8.reshape(n_edges, _PAD)
            aggp = _sc_scatter(messages, dst64, n_nodes, n_edges)
        agg = aggp.reshape(n_nodes, _PAD)
        h_pad, out_t = _gru_step(node_inputs, agg, h_pad, wia, wib, bihr,
                                 W_hh, bhhr, Wo, bor)
        outputs.append(out_t)
    return jnp.stack(outputs, axis=0)


# speedup vs baseline: 17.3332x; 1.0518x over previous
import functools
import jax
import jax.numpy as jnp
from jax import lax
from jax.experimental import pallas as pl
from jax.experimental.pallas import tpu as pltpu
from jax.experimental.pallas import tpu_sc as plsc

_N_ITERS = 7
_NF = 10
_NI = 9
_EF = 11
_NO = 9
_H1 = 96
_PAD = 16

_EB = 12800
_NB = 4000

_NC = 2
_NS = 16
_NW = _NC * _NS
_GT = 125
_GK = 16
_GCH = _GT * _GK

_INTERPRET = False


def _sc_gather(h_pad, src2d, dst2d, n_edges):
    per_w = n_edges // _NW
    n_chunks = per_w // _GCH
    rows_per_chunk = _GK
    mesh = plsc.VectorSubcoreMesh(core_axis_name="c", subcore_axis_name="s")

    @functools.partial(
        pl.kernel,
        mesh=mesh,
        compiler_params=pltpu.CompilerParams(use_tc_tiling_on_sc=False),
        out_type=[
            jax.ShapeDtypeStruct((n_edges, _PAD), jnp.float32),
            jax.ShapeDtypeStruct((n_edges, _PAD), jnp.float32),
        ],
        scratch_types=[
            pltpu.VMEM((2, _GK, _GT), jnp.int32),
            pltpu.VMEM((2, _GCH, _PAD), jnp.float32),
            pltpu.SemaphoreType.DMA,
            pltpu.SemaphoreType.DMA,
            pltpu.SemaphoreType.DMA,
        ],
    )
    def gather_kernel(h_hbm, src_hbm, dst_hbm, osrc_hbm, odst_hbm,
                      idxv, rows, semi, semg, semo):
        wid = lax.axis_index("s") * _NC + lax.axis_index("c")
        row_base = wid * (per_w // _GT)
        out_base = wid * per_w

        def run(idx_hbm, out_hbm):
            pltpu.async_copy(idx_hbm.at[pl.ds(row_base, rows_per_chunk)],
                             idxv.at[0], semi)

            def chunk(ci, carry):
                b = lax.rem(ci, 2)
                pltpu.make_async_copy(
                    idx_hbm.at[pl.ds(row_base, rows_per_chunk)],
                    idxv.at[b], semi).wait()

                @pl.when(ci + 1 < n_chunks)
                def _():
                    pltpu.async_copy(
                        idx_hbm.at[pl.ds(row_base + (ci + 1) * rows_per_chunk,
                                         rows_per_chunk)],
                        idxv.at[1 - b], semi)

                @pl.when(ci >= 2)
                def _():
                    pltpu.make_async_copy(
                        rows.at[b],
                        out_hbm.at[pl.ds(out_base, _GCH)], semo).wait()

                cps = [
                    pltpu.async_copy(h_hbm.at[idxv.at[b, j]],
                                     rows.at[b, pl.ds(j * _GT, _GT)], semg)
                    for j in range(_GK)
                ]
                for c in cps:
                    c.wait()
                pltpu.async_copy(rows.at[b],
                                 out_hbm.at[pl.ds(out_base + ci * _GCH, _GCH)],
                                 semo)
                return carry
            lax.fori_loop(0, n_chunks, chunk, 0)
            for _ in range(2 if n_chunks >= 2 else 1):
                pltpu.make_async_copy(
                    rows.at[0], out_hbm.at[pl.ds(out_base, _GCH)], semo).wait()

        run(src_hbm, osrc_hbm)
        run(dst_hbm, odst_hbm)

    return gather_kernel(h_pad, src2d, dst2d)


_ST = 64
_SR = 8
_SCH = _ST * _SR
_HALF = 50000
_TRASH = 2048
_AGGR = _HALF + _TRASH


def _sc_scatter(messages, dst64, n_nodes, n_edges, msg0=None):
    const_mode = messages is None
    n_chunks = n_edges // _SCH
    k_max = (n_chunks + _NS - 1) // _NS
    zb = 2000
    n_zchunks = _AGGR // zb
    n_ochunks = _HALF // zb
    mesh = plsc.VectorSubcoreMesh(core_axis_name="c", subcore_axis_name="s")

    @functools.partial(
        pl.kernel,
        mesh=mesh,
        compiler_params=pltpu.CompilerParams(use_tc_tiling_on_sc=False),
        out_type=jax.ShapeDtypeStruct((_NC, _HALF, _PAD), jnp.float32),
        scratch_types=[
            pltpu.VMEM((2, _SR, _ST), jnp.int32),
            pltpu.VMEM((2, _SR, _ST), jnp.int32),
            pltpu.VMEM((2, _SCH, _PAD), jnp.float32),
            pltpu.VMEM((zb, _PAD), jnp.float32),
            pltpu.VMEM_SHARED((_AGGR, _PAD), jnp.float32),
            pltpu.SemaphoreType.DMA,
            pltpu.SemaphoreType.DMA,
            pltpu.SemaphoreType.DMA,
            pltpu.SemaphoreType.DMA,
        ],
    )
    def scatter_kernel(msg_hbm, dst_hbm, out_hbm, idxv, idxw, msgb, obuf,
                       aggsh, semi, semm, sema, semo):
        cid = lax.axis_index("c")
        sid = lax.axis_index("s")
        lo = cid * _HALF

        def zrow(i, carry):
            obuf[i, :] = jnp.zeros((_PAD,), jnp.float32)
            return carry
        lax.fori_loop(0, zb, zrow, 0)

        def zchunk(k, carry):
            c = sid + _NS * k

            @pl.when(c < n_zchunks)
            def _():
                pltpu.sync_copy(obuf, aggsh.at[pl.ds(c * zb, zb)])
            return carry
        lax.fori_loop(0, (n_zchunks + _NS - 1) // _NS, zchunk, 0)
        plsc.subcore_barrier()

        if const_mode:
            pltpu.sync_copy(msg_hbm, msgb.at[0, pl.ds(0, 1)])
            rowf = msgb[0, 0, :]

            def frow(i, carry):
                msgb[0, i, :] = rowf
                msgb[1, i, :] = rowf
                return carry
            lax.fori_loop(0, _SCH, frow, 0)

        pltpu.async_copy(dst_hbm.at[pl.ds(sid * _SR, _SR)], idxv.at[0], semi)
        if not const_mode:
            pltpu.async_copy(msg_hbm.at[pl.ds(sid * _SCH, _SCH)], msgb.at[0],
                             semm)

        def chunk(k, carry):
            c = sid + _NS * k
            b = lax.rem(k, 2)

            @pl.when(c < n_chunks)
            def _():
                pltpu.make_async_copy(dst_hbm.at[pl.ds(sid * _SR, _SR)],
                                      idxv.at[b], semi).wait()
                if not const_mode:
                    pltpu.make_async_copy(
                        msg_hbm.at[pl.ds(sid * _SCH, _SCH)],
                        msgb.at[b], semm).wait()

                @pl.when(k >= 1)
                def _():
                    for j in range(_SR):
                        pltpu.make_async_copy(
                            msgb.at[1 - b, pl.ds(j * _ST, _ST)],
                            aggsh.at[idxw.at[1 - b, 0]], sema).wait()

                cn = c + _NS

                @pl.when(cn < n_chunks)
                def _():
                    pltpu.async_copy(dst_hbm.at[pl.ds(cn * _SR, _SR)],
                                     idxv.at[1 - b], semi)
                    if not const_mode:
                        pltpu.async_copy(msg_hbm.at[pl.ds(cn * _SCH, _SCH)],
                                         msgb.at[1 - b], semm)

                for j in range(_SR):
                    for t in range(_ST // 16):
                        v = idxv[b, j, pl.ds(t * 16, 16)]
                        local = v - lo
                        ok = (local >= 0) & (local < _HALF)
                        idxw[b, j, pl.ds(t * 16, 16)] = jnp.where(
                            ok, local, _HALF + (v & (_TRASH - 1)))

                for j in range(_SR):
                    pltpu.async_copy(msgb.at[b, pl.ds(j * _ST, _ST)],
                                     aggsh.at[idxw.at[b, j]], sema, add=True)
            return carry
        lax.fori_loop(0, k_max, chunk, 0)
        for j in range(_SR):
            pltpu.make_async_copy(msgb.at[0, pl.ds(j * _ST, _ST)],
                                  aggsh.at[idxw.at[0, 0]], sema).wait()
        plsc.subcore_barrier()

        def ochunk(k, carry):
            c = sid + _NS * k

            @pl.when(c < n_ochunks)
            def _():
                pltpu.sync_copy(aggsh.at[pl.ds(c * zb, zb)], obuf)
                pltpu.sync_copy(obuf, out_hbm.at[cid].at[pl.ds(c * zb, zb)])
            return carry
        lax.fori_loop(0, (n_ochunks + _NS - 1) // _NS, ochunk, 0)

    if const_mode:
        return scatter_kernel(msg0, dst64)
    return scatter_kernel(messages, dst64)


_EBP = 4000


def _mlp_body(xs_ref, xd_ref, w1s_ref, w1d_ref, b1_ref, w2_ref, b2_ref,
              w3_ref, b3_ref, out_ref):
    dn = (((1,), (1,)), ((), ()))
    m = jnp.maximum(
        jax.lax.dot_general(xs_ref[...], w1s_ref[...], dn,
                            preferred_element_type=jnp.float32)
        + jax.lax.dot_general(xd_ref[...], w1d_ref[...], dn,
                              preferred_element_type=jnp.float32)
        + b1_ref[...], 0.0)
    m = jnp.maximum(
        jax.lax.dot_general(m, w2_ref[...], dn,
                            preferred_element_type=jnp.float32) + b2_ref[...],
        0.0)
    out_ref[...] = jax.lax.dot_general(
        m, w3_ref[...], dn, preferred_element_type=jnp.float32) + b3_ref[...]


def _edge_mlp(xs, xd, w1s_bd, w1d_bd, b1t, w2_bd, b2t, w3_bd, b3t):
    n_rows = xs.shape[0]
    grid = n_rows // _EBP
    h8 = 8 * _H1
    return pl.pallas_call(
        _mlp_body,
        grid=(grid,),
        in_specs=[
            pl.BlockSpec((_EBP, 128), lambda i: (i, 0)),
            pl.BlockSpec((_EBP, 128), lambda i: (i, 0)),
            pl.BlockSpec((h8, 128), lambda i: (0, 0)),
            pl.BlockSpec((h8, 128), lambda i: (0, 0)),
            pl.BlockSpec((1, h8), lambda i: (0, 0)),
            pl.BlockSpec((h8, h8), lambda i: (0, 0)),
            pl.BlockSpec((1, h8), lambda i: (0, 0)),
            pl.BlockSpec((128, h8), lambda i: (0, 0)),
            pl.BlockSpec((1, 128), lambda i: (0, 0)),
        ],
        out_specs=pl.BlockSpec((_EBP, 128), lambda i: (i, 0)),
        out_shape=jax.ShapeDtypeStruct((n_rows, 128), jnp.float32),
        interpret=_INTERPRET,
    )(xs, xd, w1s_bd, w1d_bd, b1t, w2_bd, b2t, w3_bd, b3t)


def _gru_body(ni_ref, agg_ref, h_ref, wia_ref, wib_ref, bih_ref,
              whh_ref, bhh_ref, wo_ref, bo_ref, hout_ref, out_ref):
    dn = (((1,), (1,)), ((), ()))
    ni = ni_ref[...]
    agg = agg_ref[...]
    h = h_ref[...][:, :_NF]
    gi = (jax.lax.dot_general(ni, wia_ref[...], dn,
                              preferred_element_type=jnp.float32)
          + jax.lax.dot_general(agg, wib_ref[...], dn,
                                preferred_element_type=jnp.float32)
          + bih_ref[...])
    gh = jax.lax.dot_general(h, whh_ref[...], dn,
                             preferred_element_type=jnp.float32) + bhh_ref[...]
    i_r = gi[:, :_NF]
    i_z = gi[:, _NF:2 * _NF]
    i_n = gi[:, 2 * _NF:]
    h_r = gh[:, :_NF]
    h_z = gh[:, _NF:2 * _NF]
    h_n = gh[:, 2 * _NF:]
    r = jax.nn.sigmoid(i_r + h_r)
    z = jax.nn.sigmoid(i_z + h_z)
    n = jnp.tanh(i_n + r * h_n)
    hn = (1.0 - z) * n + z * h
    hout_ref[...] = jnp.pad(hn, ((0, 0), (0, _PAD - _NF)))
    out_ref[...] = jax.lax.dot_general(
        hn, wo_ref[...], dn, preferred_element_type=jnp.float32) + bo_ref[...]


def _gru_step(node_inputs, aggp, h_pad, wia, wib, b_ih, w_hh, b_hh, wo, bo):
    n_nodes = node_inputs.shape[0]
    grid = n_nodes // _NB
    return pl.pallas_call(
        _gru_body,
        grid=(grid,),
        in_specs=[
            pl.BlockSpec((_NB, _NI), lambda i: (i, 0)),
            pl.BlockSpec((_NB, _PAD), lambda i: (i, 0)),
            pl.BlockSpec((_NB, _PAD), lambda i: (i, 0)),
            pl.BlockSpec((3 * _NF, _NI), lambda i: (0, 0)),
            pl.BlockSpec((3 * _NF, _PAD), lambda i: (0, 0)),
            pl.BlockSpec((1, 3 * _NF), lambda i: (0, 0)),
            pl.BlockSpec((3 * _NF, _NF), lambda i: (0, 0)),
            pl.BlockSpec((1, 3 * _NF), lambda i: (0, 0)),
            pl.BlockSpec((_NO, _NF), lambda i: (0, 0)),
            pl.BlockSpec((1, _NO), lambda i: (0, 0)),
        ],
        out_specs=[
            pl.BlockSpec((_NB, _PAD), lambda i: (i, 0)),
            pl.BlockSpec((_NB, _NO), lambda i: (i, 0)),
        ],
        out_shape=[
            jax.ShapeDtypeStruct((n_nodes, _PAD), jnp.float32),
            jax.ShapeDtypeStruct((n_nodes, _NO), jnp.float32),
        ],
        interpret=_INTERPRET,
    )(node_inputs, aggp, h_pad, wia, wib, b_ih, w_hh, b_hh, wo, bo)


def kernel(node_inputs, src_ids, dst_ids, W1, b1, W2, b2, W3, b3,
           W_ih, b_ih, W_hh, b_hh, Wo, bo):
    n_nodes = node_inputs.shape[0]
    n_edges = src_ids.shape[0]

    src2d = src_ids.reshape(n_edges // _GT, _GT)
    dst2d = dst_ids.reshape(n_edges // _GT, _GT)
    dst64 = dst_ids.reshape(n_edges // _ST, _ST)

    w1s = jnp.zeros((_H1, _PAD), jnp.float32).at[:, :_NF].set(W1[:, :_NF])
    w1d = jnp.zeros((_H1, _PAD), jnp.float32).at[:, :_NF].set(W1[:, _NF:])
    w3p = jnp.zeros((_PAD, _H1), jnp.float32).at[:_EF, :].set(W3)
    b3p = jnp.zeros((_PAD,), jnp.float32).at[:_EF].set(b3)
    wia = W_ih[:, :_NI]
    wib = jnp.zeros((3 * _NF, _PAD), jnp.float32).at[:, :_EF].set(W_ih[:, _NI:])

    eye8 = jnp.eye(8, dtype=jnp.float32)
    w1s_bd = jnp.einsum('ab,ij->aibj', eye8, w1s).reshape(8 * _H1, 128)
    w1d_bd = jnp.einsum('ab,ij->aibj', eye8, w1d).reshape(8 * _H1, 128)
    w2_bd = jnp.einsum('ab,ij->aibj', eye8, W2).reshape(8 * _H1, 8 * _H1)
    w3_bd = jnp.einsum('ab,ij->aibj', eye8, w3p).reshape(128, 8 * _H1)
    b1t = jnp.tile(b1, 8).reshape(1, -1)
    b2t = jnp.tile(b2, 8).reshape(1, -1)
    b3t = jnp.tile(b3p, 8).reshape(1, -1)

    b1r = b1.reshape(1, -1)
    b2r = b2.reshape(1, -1)
    bihr = b_ih.reshape(1, -1)
    bhhr = b_hh.reshape(1, -1)
    bor = bo.reshape(1, -1)

    m0 = jnp.maximum(b1, 0.0)
    m0 = jnp.maximum(m0 @ W2.T + b2, 0.0)
    msg0 = (m0 @ w3p.T + b3p).reshape(1, _PAD)

    h_pad = jnp.zeros((n_nodes, _PAD), jnp.float32)
    outputs = []
    for it in range(_N_ITERS):
        if it == 0:
            aggp = _sc_scatter(None, dst64, n_nodes, n_edges, msg0=msg0)
        else:
            xs, xd = _sc_gather(h_pad, src2d, dst2d, n_edges)
            xs8 = xs.reshape(n_edges // 8, 128)
            xd8 = xd.reshape(n_edges // 8, 128)
            msg8 = _edge_mlp(xs8, xd8, w1s_bd, w1d_bd, b1t, w2_bd, b2t,
                             w3_bd, b3t)
            messages = msg8.reshape(n_edges, _PAD)
            aggp = _sc_scatter(messages, dst64, n_nodes, n_edges)
        agg = aggp.reshape(n_nodes, _PAD)
        h_pad, out_t = _gru_step(node_inputs, agg, h_pad, wia, wib, bihr,
                                 W_hh, bhhr, Wo, bor)
        outputs.append(out_t)
    return jnp.stack(outputs, axis=0)
